# baseline (speedup 1.0000x reference)
import functools

import jax
import jax.numpy as jnp
from jax import lax
from jax.experimental import pallas as pl
from jax.experimental.pallas import tpu as pltpu
from jax.experimental.pallas import tpu_sc as plsc

N_TOK = 16384
N_CODES = 8192
DIM = 256

BM = 512
BN = 1024
N_CHUNKS = N_CODES // BN
N_STEPS = N_TOK // BM

SC_CORES = 2
SC_SUBCORES = 16
NW = SC_CORES * SC_SUBCORES
ROWS_PER_W = N_TOK // NW
GCH = 128


def _vq_argmin_body(z_ref, w_ref, idx_ref, counts_ref, perp_ref, wsq_ref):
    i = pl.program_id(0)

    @pl.when(i == 0)
    def _():
        for jb in range(N_CHUNKS):
            w_blk = w_ref[jb * BN:(jb + 1) * BN, :]
            wsq_ref[0, jb * BN:(jb + 1) * BN] = jnp.sum(w_blk * w_blk, axis=1)

    z_blk = z_ref[...]
    zsq = jnp.sum(z_blk * z_blk, axis=1, keepdims=True)

    best_d = [jnp.full((BM, 1), jnp.inf, jnp.float32) for _ in range(2)]
    best_i = [jnp.zeros((BM, 1), jnp.int32) for _ in range(2)]
    colf = lax.broadcasted_iota(jnp.int32, (BM, BN), 1).astype(jnp.float32)
    for jb in range(N_CHUNKS):
        t = jb // (N_CHUNKS // 2)
        w_blk = w_ref[jb * BN:(jb + 1) * BN, :]
        wsq_row = wsq_ref[0:1, jb * BN:(jb + 1) * BN]
        mm = lax.dot_general(z_blk.astype(jnp.bfloat16),
                             w_blk.astype(jnp.bfloat16),
                             (((1,), (1,)), ((), ())),
                             preferred_element_type=jnp.float32)
        d = (zsq + wsq_row) - 2.0 * mm
        loc_min = jnp.min(d, axis=1, keepdims=True)
        cand = jnp.where(d == loc_min, colf, jnp.float32(2.0**30))
        loc_arg = jnp.min(cand, axis=1, keepdims=True).astype(jnp.int32) \
            + jb * BN
        upd = loc_min < best_d[t]
        best_i[t] = jnp.where(upd, loc_arg, best_i[t])
        best_d[t] = jnp.where(upd, loc_min, best_d[t])
    bf0 = best_d[0].astype(jnp.bfloat16).astype(jnp.float32)
    steal = best_d[1] < bf0
    best_i = jnp.where(steal, best_i[1], best_i[0])
    idx_ref[0] = best_i

    @pl.when(i == 0)
    def _():
        counts_ref[...] = jnp.zeros((1, N_CODES), jnp.float32)

    best_if = best_i.astype(jnp.float32)
    for jb in range(N_CHUNKS):
        cnt = jnp.sum((best_if - jb * BN == colf).astype(jnp.float32),
                      axis=0, keepdims=True)
        counts_ref[:, jb * BN:(jb + 1) * BN] += cnt

    @pl.when(i == N_STEPS - 1)
    def _():
        e_mean = counts_ref[...] * (1.0 / N_TOK)
        ent = jnp.sum(e_mean * jnp.log(e_mean + 1e-10), keepdims=True)
        perp_ref[...] = jnp.exp(-ent)


_vq_call = pl.pallas_call(
    _vq_argmin_body,
    grid=(N_STEPS,),
    in_specs=[
        pl.BlockSpec((BM, DIM), lambda i: (i, 0)),
        pl.BlockSpec((N_CODES, DIM), lambda i: (0, 0)),
    ],
    out_specs=[
        pl.BlockSpec((1, BM, 1), lambda i: (i, 0, 0)),
        pl.BlockSpec((1, N_CODES), lambda i: (0, 0)),
        pl.BlockSpec((1, 1), lambda i: (0, 0)),
    ],
    out_shape=[
        jax.ShapeDtypeStruct((N_STEPS, BM, 1), jnp.int32),
        jax.ShapeDtypeStruct((1, N_CODES), jnp.float32),
        jax.ShapeDtypeStruct((1, 1), jnp.float32),
    ],
    scratch_shapes=[pltpu.VMEM((1, N_CODES), jnp.float32)],
    compiler_params=pltpu.CompilerParams(
        dimension_semantics=("arbitrary",)),
)


@functools.cache
def _make_gather():
    mesh = plsc.VectorSubcoreMesh(core_axis_name="c", subcore_axis_name="s",
                                  num_cores=SC_CORES)

    @functools.partial(
        pl.kernel, mesh=mesh,
        out_type=jax.ShapeDtypeStruct((N_TOK, DIM), jnp.float32),
        scratch_types=[
            pltpu.VMEM((GCH,), jnp.int32),
            pltpu.VMEM((GCH, DIM), jnp.float32),
            pltpu.SemaphoreType.DMA,
        ],
    )
    def gather_k(w_hbm, idx_hbm, out_hbm, idx_v, rows_v, sem):
        wid = lax.axis_index("s") * SC_CORES + lax.axis_index("c")
        base = wid * ROWS_PER_W
        for c in range(ROWS_PER_W // GCH):
            off = base + c * GCH
            pltpu.sync_copy(idx_hbm.at[pl.ds(off, GCH)], idx_v)
            pltpu.async_copy(w_hbm.at[idx_v], rows_v, sem).wait()
            pltpu.sync_copy(rows_v, out_hbm.at[pl.ds(off, GCH)])

    return gather_k


def kernel(z, W):
    idx3, _, perp = _vq_call(z, W)
    idx = idx3.reshape(N_TOK)
    zq = _make_gather()(W, idx)
    z_q = z + lax.stop_gradient(zq - z)
    return (z_q, perp.reshape(()))


# speedup vs baseline: 1.0521x; 1.0521x over previous
import functools

import jax
import jax.numpy as jnp
from jax import lax
from jax.experimental import pallas as pl
from jax.experimental.pallas import tpu as pltpu
from jax.experimental.pallas import tpu_sc as plsc

N_TOK = 16384
N_CODES = 8192
DIM = 256

BM = 1024
BN = 1024
N_CHUNKS = N_CODES // BN
N_STEPS = N_TOK // BM

SC_CORES = 2
SC_SUBCORES = 16
NW = SC_CORES * SC_SUBCORES
ROWS_PER_W = N_TOK // NW
GCH = 128


def _vq_argmin_body(z_ref, w_ref, idx_ref, counts_ref, perp_ref, wsq_ref):
    i = pl.program_id(0)

    @pl.when(i == 0)
    def _():
        for jb in range(N_CHUNKS):
            w_blk = w_ref[jb * BN:(jb + 1) * BN, :]
            wsq_ref[0, jb * BN:(jb + 1) * BN] = jnp.sum(w_blk * w_blk, axis=1)

    z_blk = z_ref[...]
    zsq = jnp.sum(z_blk * z_blk, axis=1, keepdims=True)

    best_d = [jnp.full((BM, 1), jnp.inf, jnp.float32) for _ in range(2)]
    best_i = [jnp.zeros((BM, 1), jnp.int32) for _ in range(2)]
    colf = lax.broadcasted_iota(jnp.int32, (BM, BN), 1).astype(jnp.float32)
    for jb in range(N_CHUNKS):
        t = jb // (N_CHUNKS // 2)
        w_blk = w_ref[jb * BN:(jb + 1) * BN, :]
        wsq_row = wsq_ref[0:1, jb * BN:(jb + 1) * BN]
        mm = lax.dot_general(z_blk, w_blk, (((1,), (1,)), ((), ())),
                             preferred_element_type=jnp.float32)
        d = (zsq + wsq_row) - 2.0 * mm
        loc_min = jnp.min(d, axis=1, keepdims=True)
        cand = jnp.where(d == loc_min, colf, jnp.float32(2.0**30))
        loc_arg = jnp.min(cand, axis=1, keepdims=True).astype(jnp.int32) \
            + jb * BN
        upd = loc_min < best_d[t]
        best_i[t] = jnp.where(upd, loc_arg, best_i[t])
        best_d[t] = jnp.where(upd, loc_min, best_d[t])
    bf0 = best_d[0].astype(jnp.bfloat16).astype(jnp.float32)
    steal = best_d[1] < bf0
    best_i = jnp.where(steal, best_i[1], best_i[0])
    idx_ref[0] = best_i

    @pl.when(i == 0)
    def _():
        counts_ref[...] = jnp.zeros((1, N_CODES), jnp.float32)

    best_if = best_i.astype(jnp.float32)
    for jb in range(N_CHUNKS):
        cnt = jnp.sum((best_if - jb * BN == colf).astype(jnp.float32),
                      axis=0, keepdims=True)
        counts_ref[:, jb * BN:(jb + 1) * BN] += cnt

    @pl.when(i == N_STEPS - 1)
    def _():
        e_mean = counts_ref[...] * (1.0 / N_TOK)
        ent = jnp.sum(e_mean * jnp.log(e_mean + 1e-10), keepdims=True)
        perp_ref[...] = jnp.exp(-ent)


_vq_call = pl.pallas_call(
    _vq_argmin_body,
    grid=(N_STEPS,),
    in_specs=[
        pl.BlockSpec((BM, DIM), lambda i: (i, 0)),
        pl.BlockSpec((N_CODES, DIM), lambda i: (0, 0)),
    ],
    out_specs=[
        pl.BlockSpec((1, BM, 1), lambda i: (i, 0, 0)),
        pl.BlockSpec((1, N_CODES), lambda i: (0, 0)),
        pl.BlockSpec((1, 1), lambda i: (0, 0)),
    ],
    out_shape=[
        jax.ShapeDtypeStruct((N_STEPS, BM, 1), jnp.int32),
        jax.ShapeDtypeStruct((1, N_CODES), jnp.float32),
        jax.ShapeDtypeStruct((1, 1), jnp.float32),
    ],
    scratch_shapes=[pltpu.VMEM((1, N_CODES), jnp.float32)],
    compiler_params=pltpu.CompilerParams(
        dimension_semantics=("arbitrary",)),
)


@functools.cache
def _make_gather():
    mesh = plsc.VectorSubcoreMesh(core_axis_name="c", subcore_axis_name="s",
                                  num_cores=SC_CORES)

    @functools.partial(
        pl.kernel, mesh=mesh,
        out_type=jax.ShapeDtypeStruct((N_TOK, DIM), jnp.float32),
        scratch_types=[
            pltpu.VMEM((GCH,), jnp.int32),
            pltpu.VMEM((GCH, DIM), jnp.float32),
            pltpu.SemaphoreType.DMA,
        ],
    )
    def gather_k(w_hbm, idx_hbm, out_hbm, idx_v, rows_v, sem):
        wid = lax.axis_index("s") * SC_CORES + lax.axis_index("c")
        base = wid * ROWS_PER_W
        for c in range(ROWS_PER_W // GCH):
            off = base + c * GCH
            pltpu.sync_copy(idx_hbm.at[pl.ds(off, GCH)], idx_v)
            pltpu.async_copy(w_hbm.at[idx_v], rows_v, sem).wait()
            pltpu.sync_copy(rows_v, out_hbm.at[pl.ds(off, GCH)])

    return gather_k


def kernel(z, W):
    idx3, _, perp = _vq_call(z, W)
    idx = idx3.reshape(N_TOK)
    zq = _make_gather()(W, idx)
    z_q = z + lax.stop_gradient(zq - z)
    return (z_q, perp.reshape(()))
